# Initial kernel scaffold; baseline (speedup 1.0000x reference)
import jax
import jax.numpy as jnp
from jax.experimental import pallas as pl


def kernel(node_features, edge_index, W1, b1, W2, b2, W3, b3, W4, b4, W5, b5):
    raise NotImplementedError("write your pallas kernel here")


# speedup vs baseline: 9.6851x; 9.6851x over previous
import functools

import jax
import jax.numpy as jnp
from jax import lax
from jax.experimental import pallas as pl
from jax.experimental.pallas import tpu as pltpu
from jax.experimental.pallas import tpu_sc as plsc

N = 10000
E = 160000
D = 256
HALF = 128
NC = 2
NS = 16
ROWS_PER_TILE = N // NS
RCHUNK = 632
RLAST = N - (NS - 1) * RCHUNK
EDGES_PER_TILE_SC = E // NS
EB = 80
EDGES_PER_TILE_ALL = E // (NC * NS)
MBLK = 1000
GRID_M = N // MBLK

_mesh = plsc.VectorSubcoreMesh(core_axis_name="c", subcore_axis_name="s")


@functools.partial(
    pl.kernel,
    out_type=jax.ShapeDtypeStruct((NC * NS * N,), jnp.float32),
    mesh=_mesh,
    compiler_params=pltpu.CompilerParams(needs_layout_passes=False),
    scratch_types=[
        pltpu.VMEM((N,), jnp.float32),
        pltpu.VMEM((EDGES_PER_TILE_ALL + 16,), jnp.int32),
    ],
)
def _deg_kernel(col_hbm, out_hbm, hist, colbuf):
    c = lax.axis_index("c")
    s = lax.axis_index("s")
    wid = s * NC + c
    zeros16 = jnp.zeros((16,), jnp.float32)

    def zbody(j, carry):
        hist[pl.ds(j * 16, 16)] = zeros16
        return carry

    lax.fori_loop(0, N // 16, zbody, 0)
    pltpu.sync_copy(
        col_hbm.at[pl.ds(wid * EDGES_PER_TILE_ALL, EDGES_PER_TILE_ALL)],
        colbuf.at[pl.ds(0, EDGES_PER_TILE_ALL)],
    )
    ones16 = jnp.ones((16,), jnp.float32)
    lanes = lax.iota(jnp.int32, 16)
    nit = (EDGES_PER_TILE_ALL + 15) // 16

    def body(j, carry):
        idx = colbuf[pl.ds(j * 16, 16)]
        idx = jnp.minimum(jnp.maximum(idx, 0), N - 1)
        valid = lanes < (EDGES_PER_TILE_ALL - j * 16)
        vals = jnp.where(valid, ones16, 0.0)
        plsc.addupdate_scatter(hist, [idx], vals)
        return carry

    lax.fori_loop(0, nit, body, 0)
    pltpu.sync_copy(hist, out_hbm.at[pl.ds(wid * N, N)])


@functools.partial(
    pl.kernel,
    out_type=(
        jax.ShapeDtypeStruct((N, HALF), jnp.float32),
        jax.ShapeDtypeStruct((N, HALF), jnp.float32),
    ),
    mesh=_mesh,
    compiler_params=pltpu.CompilerParams(needs_layout_passes=False),
    scratch_types=[
        pltpu.VMEM_SHARED((N, HALF), jnp.float32),
        pltpu.VMEM((EB,), jnp.int32),
        pltpu.VMEM((EB,), jnp.int32),
        pltpu.VMEM((EB, HALF), jnp.float32),
        pltpu.SemaphoreType.DMA,
    ],
)
def _prop_kernel(y0_hbm, y1_hbm, row_hbm, col_hbm, out0_hbm, out1_hbm,
                 accs, rowbuf, colbuf, rowsbuf, sem):
    c = lax.axis_index("c")
    s = lax.axis_index("s")

    def run(y_hbm, out_hbm):
        base = s * RCHUNK
        nsl_a = pl.ds(base, RLAST)
        nsl_b = pl.ds(base + RLAST, RCHUNK - RLAST)
        pltpu.sync_copy(y_hbm.at[nsl_a], accs.at[nsl_a])
        pl.when(s < NS - 1)(
            lambda: pltpu.sync_copy(y_hbm.at[nsl_b], accs.at[nsl_b]))
        plsc.subcore_barrier()

        def body(j, carry):
            off = s * EDGES_PER_TILE_SC + j * EB
            pltpu.sync_copy(row_hbm.at[pl.ds(off, EB)], rowbuf)
            pltpu.sync_copy(col_hbm.at[pl.ds(off, EB)], colbuf)
            pltpu.async_copy(y_hbm.at[rowbuf], rowsbuf, sem).wait()
            pltpu.sync_copy(rowsbuf, accs.at[colbuf], add=True)
            return carry

        lax.fori_loop(0, EDGES_PER_TILE_SC // EB, body, 0)
        plsc.subcore_barrier()
        pltpu.sync_copy(accs.at[nsl_a], out_hbm.at[nsl_a])
        pl.when(s < NS - 1)(
            lambda: pltpu.sync_copy(accs.at[nsl_b], out_hbm.at[nsl_b]))

    pl.when(c == 0)(lambda: run(y0_hbm, out0_hbm))
    pl.when(c == 1)(lambda: run(y1_hbm, out1_hbm))


@functools.partial(
    pl.kernel,
    out_type=jax.ShapeDtypeStruct((NC * NS * N,), jnp.float32),
    mesh=_mesh,
    compiler_params=pltpu.CompilerParams(needs_layout_passes=False),
    scratch_types=[
        pltpu.VMEM((N,), jnp.float32),
        pltpu.VMEM((N,), jnp.float32),
        pltpu.VMEM((EDGES_PER_TILE_ALL + 16,), jnp.int32),
        pltpu.VMEM((EDGES_PER_TILE_ALL + 16,), jnp.int32),
    ],
)
def _prop1_kernel(y3_hbm, row_hbm, col_hbm, out_hbm, acc, y3v, rowbuf, colbuf):
    c = lax.axis_index("c")
    s = lax.axis_index("s")
    wid = s * NC + c
    pltpu.sync_copy(y3_hbm, y3v)
    zeros16 = jnp.zeros((16,), jnp.float32)

    def zbody(j, carry):
        acc[pl.ds(j * 16, 16)] = zeros16
        return carry

    lax.fori_loop(0, N // 16, zbody, 0)
    esl = pl.ds(wid * EDGES_PER_TILE_ALL, EDGES_PER_TILE_ALL)
    bsl = pl.ds(0, EDGES_PER_TILE_ALL)
    pltpu.sync_copy(row_hbm.at[esl], rowbuf.at[bsl])
    pltpu.sync_copy(col_hbm.at[esl], colbuf.at[bsl])
    lanes = lax.iota(jnp.int32, 16)
    nit = (EDGES_PER_TILE_ALL + 15) // 16

    def body(j, carry):
        ridx = jnp.minimum(jnp.maximum(rowbuf[pl.ds(j * 16, 16)], 0), N - 1)
        cidx = jnp.minimum(jnp.maximum(colbuf[pl.ds(j * 16, 16)], 0), N - 1)
        valid = lanes < (EDGES_PER_TILE_ALL - j * 16)
        vals = jnp.where(valid, plsc.load_gather(y3v, [ridx]), 0.0)
        plsc.addupdate_scatter(acc, [cidx], vals)
        return carry

    lax.fori_loop(0, nit, body, 0)
    pltpu.sync_copy(acc, out_hbm.at[pl.ds(wid * N, N)])


def _deg_reduce_body(p_ref, deg_ref):
    p = p_ref[...]
    ones = jnp.ones((p.shape[0], 1), jnp.float32)
    deg_ref[...] = lax.dot_general(
        p, ones, (((0,), (0,)), ((), ()))) + 1.0


_deg_reduce = pl.pallas_call(
    _deg_reduce_body,
    out_shape=jax.ShapeDtypeStruct((N, 1), jnp.float32),
)


def _mm1_body(x_ref, w_ref, deg_ref, y0_ref, y1_ref):
    dis = lax.rsqrt(deg_ref[...])
    y = jnp.dot(x_ref[...], w_ref[...],
                preferred_element_type=jnp.float32) * dis
    y0_ref[...] = y[:, :HALF]
    y1_ref[...] = y[:, HALF:]


_mm1 = pl.pallas_call(
    _mm1_body,
    grid=(GRID_M,),
    in_specs=[
        pl.BlockSpec((MBLK, D), lambda i: (i, 0)),
        pl.BlockSpec((D, D), lambda i: (0, 0)),
        pl.BlockSpec((MBLK, 1), lambda i: (i, 0)),
    ],
    out_specs=(
        pl.BlockSpec((MBLK, HALF), lambda i: (i, 0)),
        pl.BlockSpec((MBLK, HALF), lambda i: (i, 0)),
    ),
    out_shape=(
        jax.ShapeDtypeStruct((N, HALF), jnp.float32),
        jax.ShapeDtypeStruct((N, HALF), jnp.float32),
    ),
)


def _mm2_body(a0_ref, a1_ref, deg_ref, w_ref, b_ref, y0_ref, y1_ref):
    dis = lax.rsqrt(deg_ref[...])
    b = b_ref[...]
    h0 = jax.nn.sigmoid(a0_ref[...] * dis + b[:, :HALF])
    h1 = jax.nn.sigmoid(a1_ref[...] * dis + b[:, HALF:])
    h = jnp.concatenate([h0, h1], axis=1)
    y = jnp.dot(h, w_ref[...], preferred_element_type=jnp.float32) * dis
    y0_ref[...] = y[:, :HALF]
    y1_ref[...] = y[:, HALF:]


_mm2 = pl.pallas_call(
    _mm2_body,
    grid=(GRID_M,),
    in_specs=[
        pl.BlockSpec((MBLK, HALF), lambda i: (i, 0)),
        pl.BlockSpec((MBLK, HALF), lambda i: (i, 0)),
        pl.BlockSpec((MBLK, 1), lambda i: (i, 0)),
        pl.BlockSpec((D, D), lambda i: (0, 0)),
        pl.BlockSpec((1, D), lambda i: (0, 0)),
    ],
    out_specs=(
        pl.BlockSpec((MBLK, HALF), lambda i: (i, 0)),
        pl.BlockSpec((MBLK, HALF), lambda i: (i, 0)),
    ),
    out_shape=(
        jax.ShapeDtypeStruct((N, HALF), jnp.float32),
        jax.ShapeDtypeStruct((N, HALF), jnp.float32),
    ),
)


def _mm3_body(a0_ref, a1_ref, deg_ref, w_ref, b_ref, y3_ref):
    dis = lax.rsqrt(deg_ref[...])
    b = b_ref[...]
    h0 = jax.nn.sigmoid(a0_ref[...] * dis + b[:, :HALF])
    h1 = jax.nn.sigmoid(a1_ref[...] * dis + b[:, HALF:])
    h = jnp.concatenate([h0, h1], axis=1)
    y3_ref[...] = jnp.dot(h, w_ref[...],
                          preferred_element_type=jnp.float32) * dis


_mm3 = pl.pallas_call(
    _mm3_body,
    grid=(GRID_M,),
    in_specs=[
        pl.BlockSpec((MBLK, HALF), lambda i: (i, 0)),
        pl.BlockSpec((MBLK, HALF), lambda i: (i, 0)),
        pl.BlockSpec((MBLK, 1), lambda i: (i, 0)),
        pl.BlockSpec((D, 1), lambda i: (0, 0)),
        pl.BlockSpec((1, D), lambda i: (0, 0)),
    ],
    out_specs=pl.BlockSpec((MBLK, 1), lambda i: (i, 0)),
    out_shape=jax.ShapeDtypeStruct((N, 1), jnp.float32),
)


def _final_body(p_ref, y3_ref, deg_ref, b3_ref, w4_ref, b4_ref, w5_ref,
                b5_ref, out_ref):
    p = p_ref[...]
    ones = jnp.ones((p.shape[0], 1), jnp.float32)
    acc3 = lax.dot_general(p, ones, (((0,), (0,)), ((), ()))) + y3_ref[...]
    out3 = lax.rsqrt(deg_ref[...]) * acc3 + b3_ref[...]
    m = jnp.sum(out3) * (1.0 / N)
    r = (m * w4_ref[...] + b4_ref[...]) * w5_ref[...] + b5_ref[...]
    out_ref[...] = r


_final = pl.pallas_call(
    _final_body,
    out_shape=jax.ShapeDtypeStruct((1, 1), jnp.float32),
)


def kernel(node_features, edge_index, W1, b1, W2, b2, W3, b3, W4, b4, W5, b5):
    row = edge_index[0]
    col = edge_index[1]
    partials = _deg_kernel(col).reshape(NC * NS, N)
    deg = _deg_reduce(partials)
    y10, y11 = _mm1(node_features, W1, deg)
    a10, a11 = _prop_kernel(y10, y11, row, col)
    y20, y21 = _mm2(a10, a11, deg, W2, b1.reshape(1, D))
    a20, a21 = _prop_kernel(y20, y21, row, col)
    y3 = _mm3(a20, a21, deg, W3, b2.reshape(1, D))
    p3 = _prop1_kernel(y3.reshape(N), row, col).reshape(NC * NS, N)
    out = _final(p3, y3, deg, b3.reshape(1, 1), W4, b4.reshape(1, 1),
                 W5, b5.reshape(1, 1))
    return out


# speedup vs baseline: 18.9182x; 1.9533x over previous
import functools

import jax
import jax.numpy as jnp
from jax import lax
from jax.experimental import pallas as pl
from jax.experimental.pallas import tpu as pltpu
from jax.experimental.pallas import tpu_sc as plsc

N = 10000
E = 160000
D = 256
HALF = 128
NC = 2
NS = 16
ROWS_PER_TILE = N // NS
RCHUNK = 632
RLAST = N - (NS - 1) * RCHUNK
EDGES_PER_TILE_SC = E // NS
EB = 80
EDGES_PER_TILE_ALL = E // (NC * NS)
MBLK = 1000
GRID_M = N // MBLK

_mesh = plsc.VectorSubcoreMesh(core_axis_name="c", subcore_axis_name="s")


@functools.partial(
    pl.kernel,
    out_type=jax.ShapeDtypeStruct((NC * NS * N,), jnp.float32),
    mesh=_mesh,
    compiler_params=pltpu.CompilerParams(needs_layout_passes=False),
    scratch_types=[
        pltpu.VMEM((N,), jnp.float32),
        pltpu.VMEM((EDGES_PER_TILE_ALL + 16,), jnp.int32),
    ],
)
def _deg_kernel(col_hbm, out_hbm, hist, colbuf):
    c = lax.axis_index("c")
    s = lax.axis_index("s")
    wid = s * NC + c
    zeros16 = jnp.zeros((16,), jnp.float32)

    def zbody(j, carry):
        hist[pl.ds(j * 16, 16)] = zeros16
        return carry

    lax.fori_loop(0, N // 16, zbody, 0)
    pltpu.sync_copy(
        col_hbm.at[pl.ds(wid * EDGES_PER_TILE_ALL, EDGES_PER_TILE_ALL)],
        colbuf.at[pl.ds(0, EDGES_PER_TILE_ALL)],
    )
    ones16 = jnp.ones((16,), jnp.float32)
    lanes = lax.iota(jnp.int32, 16)
    nit = (EDGES_PER_TILE_ALL + 15) // 16

    def body(j, carry):
        idx = colbuf[pl.ds(j * 16, 16)]
        idx = jnp.minimum(jnp.maximum(idx, 0), N - 1)
        valid = lanes < (EDGES_PER_TILE_ALL - j * 16)
        vals = jnp.where(valid, ones16, 0.0)
        plsc.addupdate_scatter(hist, [idx], vals)
        return carry

    lax.fori_loop(0, nit, body, 0)
    pltpu.sync_copy(hist, out_hbm.at[pl.ds(wid * N, N)])


NB = EDGES_PER_TILE_SC // EB


@functools.partial(
    pl.kernel,
    out_type=(
        jax.ShapeDtypeStruct((N, HALF), jnp.float32),
        jax.ShapeDtypeStruct((N, HALF), jnp.float32),
    ),
    mesh=_mesh,
    compiler_params=pltpu.CompilerParams(needs_layout_passes=False),
    scratch_types=[
        pltpu.VMEM_SHARED((N, HALF), jnp.float32),
        pltpu.VMEM((NB, EB), jnp.int32),
        pltpu.VMEM((EB,), jnp.int32),
        pltpu.VMEM((EB, HALF), jnp.float32),
        pltpu.VMEM((EB, HALF), jnp.float32),
        pltpu.SemaphoreType.DMA,
        pltpu.SemaphoreType.DMA,
    ],
)
def _prop_kernel(y0_hbm, y1_hbm, row_hbm, col_hbm, out0_hbm, out1_hbm,
                 accs, rowbuf, colbuf, rows0, rows1, sem0, sem1):
    c = lax.axis_index("c")
    s = lax.axis_index("s")

    def run(y_hbm, out_hbm):
        base = s * RCHUNK
        nsl_a = pl.ds(base, RLAST)
        nsl_b = pl.ds(base + RLAST, RCHUNK - RLAST)
        pltpu.sync_copy(row_hbm.at[s], rowbuf)
        pltpu.sync_copy(y_hbm.at[nsl_a], accs.at[nsl_a])
        pl.when(s < NS - 1)(
            lambda: pltpu.sync_copy(y_hbm.at[nsl_b], accs.at[nsl_b]))
        plsc.subcore_barrier()

        def start(j, buf, sem):
            pltpu.async_copy(y_hbm.at[rowbuf.at[j]], buf, sem)

        def finish(buf, sem):
            pltpu.make_async_copy(y_hbm.at[rowbuf.at[0]], buf, sem).wait()

        start(0, rows0, sem0)
        start(1, rows1, sem1)

        def body(j, carry):
            def step(buf, sem):
                pltpu.sync_copy(
                    col_hbm.at[pl.ds(s * EDGES_PER_TILE_SC + j * EB, EB)],
                    colbuf)
                finish(buf, sem)
                pltpu.sync_copy(buf, accs.at[colbuf], add=True)
                pl.when(j < NB - 2)(lambda: start(j + 2, buf, sem))
            pl.when(j % 2 == 0)(lambda: step(rows0, sem0))
            pl.when(j % 2 == 1)(lambda: step(rows1, sem1))
            return carry

        lax.fori_loop(0, NB, body, 0)
        plsc.subcore_barrier()
        pltpu.sync_copy(accs.at[nsl_a], out_hbm.at[nsl_a])
        pl.when(s < NS - 1)(
            lambda: pltpu.sync_copy(accs.at[nsl_b], out_hbm.at[nsl_b]))

    pl.when(c == 0)(lambda: run(y0_hbm, out0_hbm))
    pl.when(c == 1)(lambda: run(y1_hbm, out1_hbm))


@functools.partial(
    pl.kernel,
    out_type=jax.ShapeDtypeStruct((NC * NS * N,), jnp.float32),
    mesh=_mesh,
    compiler_params=pltpu.CompilerParams(needs_layout_passes=False),
    scratch_types=[
        pltpu.VMEM((N,), jnp.float32),
        pltpu.VMEM((N,), jnp.float32),
        pltpu.VMEM((EDGES_PER_TILE_ALL + 16,), jnp.int32),
        pltpu.VMEM((EDGES_PER_TILE_ALL + 16,), jnp.int32),
    ],
)
def _prop1_kernel(y3_hbm, row_hbm, col_hbm, out_hbm, acc, y3v, rowbuf, colbuf):
    c = lax.axis_index("c")
    s = lax.axis_index("s")
    wid = s * NC + c
    pltpu.sync_copy(y3_hbm, y3v)
    zeros16 = jnp.zeros((16,), jnp.float32)

    def zbody(j, carry):
        acc[pl.ds(j * 16, 16)] = zeros16
        return carry

    lax.fori_loop(0, N // 16, zbody, 0)
    esl = pl.ds(wid * EDGES_PER_TILE_ALL, EDGES_PER_TILE_ALL)
    bsl = pl.ds(0, EDGES_PER_TILE_ALL)
    pltpu.sync_copy(row_hbm.at[esl], rowbuf.at[bsl])
    pltpu.sync_copy(col_hbm.at[esl], colbuf.at[bsl])
    lanes = lax.iota(jnp.int32, 16)
    nit = (EDGES_PER_TILE_ALL + 15) // 16

    def body(j, carry):
        ridx = jnp.minimum(jnp.maximum(rowbuf[pl.ds(j * 16, 16)], 0), N - 1)
        cidx = jnp.minimum(jnp.maximum(colbuf[pl.ds(j * 16, 16)], 0), N - 1)
        valid = lanes < (EDGES_PER_TILE_ALL - j * 16)
        vals = jnp.where(valid, plsc.load_gather(y3v, [ridx]), 0.0)
        plsc.addupdate_scatter(acc, [cidx], vals)
        return carry

    lax.fori_loop(0, nit, body, 0)
    pltpu.sync_copy(acc, out_hbm.at[pl.ds(wid * N, N)])


def _deg_reduce_body(p_ref, deg_ref):
    p = p_ref[...]
    ones = jnp.ones((p.shape[0], 1), jnp.float32)
    deg_ref[...] = lax.dot_general(
        p, ones, (((0,), (0,)), ((), ()))) + 1.0


_deg_reduce = pl.pallas_call(
    _deg_reduce_body,
    out_shape=jax.ShapeDtypeStruct((N, 1), jnp.float32),
)


def _mm1_body(x_ref, w_ref, deg_ref, y0_ref, y1_ref):
    dis = lax.rsqrt(deg_ref[...])
    y = jnp.dot(x_ref[...], w_ref[...],
                preferred_element_type=jnp.float32) * dis
    y0_ref[...] = y[:, :HALF]
    y1_ref[...] = y[:, HALF:]


_mm1 = pl.pallas_call(
    _mm1_body,
    grid=(GRID_M,),
    in_specs=[
        pl.BlockSpec((MBLK, D), lambda i: (i, 0)),
        pl.BlockSpec((D, D), lambda i: (0, 0)),
        pl.BlockSpec((MBLK, 1), lambda i: (i, 0)),
    ],
    out_specs=(
        pl.BlockSpec((MBLK, HALF), lambda i: (i, 0)),
        pl.BlockSpec((MBLK, HALF), lambda i: (i, 0)),
    ),
    out_shape=(
        jax.ShapeDtypeStruct((N, HALF), jnp.float32),
        jax.ShapeDtypeStruct((N, HALF), jnp.float32),
    ),
)


def _mm2_body(a0_ref, a1_ref, deg_ref, w_ref, b_ref, y0_ref, y1_ref):
    dis = lax.rsqrt(deg_ref[...])
    b = b_ref[...]
    h0 = jax.nn.sigmoid(a0_ref[...] * dis + b[:, :HALF])
    h1 = jax.nn.sigmoid(a1_ref[...] * dis + b[:, HALF:])
    h = jnp.concatenate([h0, h1], axis=1)
    y = jnp.dot(h, w_ref[...], preferred_element_type=jnp.float32) * dis
    y0_ref[...] = y[:, :HALF]
    y1_ref[...] = y[:, HALF:]


_mm2 = pl.pallas_call(
    _mm2_body,
    grid=(GRID_M,),
    in_specs=[
        pl.BlockSpec((MBLK, HALF), lambda i: (i, 0)),
        pl.BlockSpec((MBLK, HALF), lambda i: (i, 0)),
        pl.BlockSpec((MBLK, 1), lambda i: (i, 0)),
        pl.BlockSpec((D, D), lambda i: (0, 0)),
        pl.BlockSpec((1, D), lambda i: (0, 0)),
    ],
    out_specs=(
        pl.BlockSpec((MBLK, HALF), lambda i: (i, 0)),
        pl.BlockSpec((MBLK, HALF), lambda i: (i, 0)),
    ),
    out_shape=(
        jax.ShapeDtypeStruct((N, HALF), jnp.float32),
        jax.ShapeDtypeStruct((N, HALF), jnp.float32),
    ),
)


def _mm3_body(a0_ref, a1_ref, deg_ref, w_ref, b_ref, y3_ref):
    dis = lax.rsqrt(deg_ref[...])
    b = b_ref[...]
    h0 = jax.nn.sigmoid(a0_ref[...] * dis + b[:, :HALF])
    h1 = jax.nn.sigmoid(a1_ref[...] * dis + b[:, HALF:])
    h = jnp.concatenate([h0, h1], axis=1)
    y3_ref[...] = jnp.dot(h, w_ref[...],
                          preferred_element_type=jnp.float32) * dis


_mm3 = pl.pallas_call(
    _mm3_body,
    grid=(GRID_M,),
    in_specs=[
        pl.BlockSpec((MBLK, HALF), lambda i: (i, 0)),
        pl.BlockSpec((MBLK, HALF), lambda i: (i, 0)),
        pl.BlockSpec((MBLK, 1), lambda i: (i, 0)),
        pl.BlockSpec((D, 1), lambda i: (0, 0)),
        pl.BlockSpec((1, D), lambda i: (0, 0)),
    ],
    out_specs=pl.BlockSpec((MBLK, 1), lambda i: (i, 0)),
    out_shape=jax.ShapeDtypeStruct((N, 1), jnp.float32),
)


def _final_body(p_ref, y3_ref, deg_ref, b3_ref, w4_ref, b4_ref, w5_ref,
                b5_ref, out_ref):
    p = p_ref[...]
    ones = jnp.ones((p.shape[0], 1), jnp.float32)
    acc3 = lax.dot_general(p, ones, (((0,), (0,)), ((), ()))) + y3_ref[...]
    out3 = lax.rsqrt(deg_ref[...]) * acc3 + b3_ref[...]
    m = jnp.sum(out3) * (1.0 / N)
    r = (m * w4_ref[...] + b4_ref[...]) * w5_ref[...] + b5_ref[...]
    out_ref[...] = r


_final = pl.pallas_call(
    _final_body,
    out_shape=jax.ShapeDtypeStruct((1, 1), jnp.float32),
)


def kernel(node_features, edge_index, W1, b1, W2, b2, W3, b3, W4, b4, W5, b5):
    row = edge_index[0]
    col = edge_index[1]
    row3 = row.reshape(NS, NB, EB)
    partials = _deg_kernel(col).reshape(NC * NS, N)
    deg = _deg_reduce(partials)
    y10, y11 = _mm1(node_features, W1, deg)
    a10, a11 = _prop_kernel(y10, y11, row3, col)
    y20, y21 = _mm2(a10, a11, deg, W2, b1.reshape(1, D))
    a20, a21 = _prop_kernel(y20, y21, row3, col)
    y3 = _mm3(a20, a21, deg, W3, b2.reshape(1, D))
    p3 = _prop1_kernel(y3.reshape(N), row, col).reshape(NC * NS, N)
    out = _final(p3, y3, deg, b3.reshape(1, 1), W4, b4.reshape(1, 1),
                 W5, b5.reshape(1, 1))
    return out


# speedup vs baseline: 20.4343x; 1.0801x over previous
import functools

import jax
import jax.numpy as jnp
from jax import lax
from jax.experimental import pallas as pl
from jax.experimental.pallas import tpu as pltpu
from jax.experimental.pallas import tpu_sc as plsc

N = 10000
E = 160000
D = 256
HALF = 128
NC = 2
NS = 16
ROWS_PER_TILE = N // NS
RCHUNK = 632
RLAST = N - (NS - 1) * RCHUNK
EDGES_PER_TILE_SC = E // NS
EB = 80
EDGES_PER_TILE_ALL = E // (NC * NS)
MBLK = 1000
GRID_M = N // MBLK

_mesh = plsc.VectorSubcoreMesh(core_axis_name="c", subcore_axis_name="s")


@functools.partial(
    pl.kernel,
    out_type=jax.ShapeDtypeStruct((NC * NS * N,), jnp.float32),
    mesh=_mesh,
    compiler_params=pltpu.CompilerParams(needs_layout_passes=False),
    scratch_types=[
        pltpu.VMEM((N,), jnp.float32),
        pltpu.VMEM((EDGES_PER_TILE_ALL + 16,), jnp.int32),
    ],
)
def _deg_kernel(col_hbm, out_hbm, hist, colbuf):
    c = lax.axis_index("c")
    s = lax.axis_index("s")
    wid = s * NC + c
    zeros16 = jnp.zeros((16,), jnp.float32)

    def zbody(j, carry):
        hist[pl.ds(j * 16, 16)] = zeros16
        return carry

    lax.fori_loop(0, N // 16, zbody, 0)
    pltpu.sync_copy(
        col_hbm.at[pl.ds(wid * EDGES_PER_TILE_ALL, EDGES_PER_TILE_ALL)],
        colbuf.at[pl.ds(0, EDGES_PER_TILE_ALL)],
    )
    ones16 = jnp.ones((16,), jnp.float32)
    lanes = lax.iota(jnp.int32, 16)
    nit = (EDGES_PER_TILE_ALL + 15) // 16

    def body(j, carry):
        idx = colbuf[pl.ds(j * 16, 16)]
        idx = jnp.minimum(jnp.maximum(idx, 0), N - 1)
        valid = lanes < (EDGES_PER_TILE_ALL - j * 16)
        vals = jnp.where(valid, ones16, 0.0)
        plsc.addupdate_scatter(hist, [idx], vals)
        return carry

    lax.fori_loop(0, nit, body, 0)
    pltpu.sync_copy(hist, out_hbm.at[pl.ds(wid * N, N)])


NB = EDGES_PER_TILE_SC // EB


@functools.partial(
    pl.kernel,
    out_type=(
        jax.ShapeDtypeStruct((N, HALF), jnp.float32),
        jax.ShapeDtypeStruct((N, HALF), jnp.float32),
    ),
    mesh=_mesh,
    compiler_params=pltpu.CompilerParams(needs_layout_passes=False),
    scratch_types=[
        pltpu.VMEM_SHARED((N, HALF), jnp.float32),
        pltpu.VMEM((NB, EB), jnp.int32),
        pltpu.VMEM((EB,), jnp.int32),
        pltpu.VMEM((EB,), jnp.int32),
        pltpu.VMEM((EB, HALF), jnp.float32),
        pltpu.VMEM((EB, HALF), jnp.float32),
        pltpu.SemaphoreType.DMA,
        pltpu.SemaphoreType.DMA,
        pltpu.SemaphoreType.DMA,
        pltpu.SemaphoreType.DMA,
    ],
)
def _prop_kernel(y0_hbm, y1_hbm, row_hbm, col_hbm, out0_hbm, out1_hbm,
                 accs, rowbuf, col0, col1, rows0, rows1,
                 sem0, sem1, csem0, csem1):
    c = lax.axis_index("c")
    s = lax.axis_index("s")

    def run(y_hbm, out_hbm):
        base = s * RCHUNK
        nsl_a = pl.ds(base, RLAST)
        nsl_b = pl.ds(base + RLAST, RCHUNK - RLAST)
        pltpu.sync_copy(row_hbm.at[s], rowbuf)
        pltpu.sync_copy(y_hbm.at[nsl_a], accs.at[nsl_a])
        pl.when(s < NS - 1)(
            lambda: pltpu.sync_copy(y_hbm.at[nsl_b], accs.at[nsl_b]))
        plsc.subcore_barrier()

        def start(j, buf, sem):
            pltpu.async_copy(y_hbm.at[rowbuf.at[j]], buf, sem)

        def finish(buf, sem):
            pltpu.make_async_copy(y_hbm.at[rowbuf.at[0]], buf, sem).wait()

        def cstart(j, cbuf, csem):
            pltpu.async_copy(
                col_hbm.at[pl.ds(s * EDGES_PER_TILE_SC + j * EB, EB)],
                cbuf, csem)

        def cfinish(cbuf, csem):
            pltpu.make_async_copy(col_hbm.at[pl.ds(0, EB)], cbuf, csem).wait()

        start(0, rows0, sem0)
        start(1, rows1, sem1)
        cstart(0, col0, csem0)
        cstart(1, col1, csem1)

        def body(j, carry):
            def step(buf, sem, cbuf, csem):
                cfinish(cbuf, csem)
                finish(buf, sem)
                pltpu.sync_copy(buf, accs.at[cbuf], add=True)
                @pl.when(j < NB - 2)
                def _():
                    start(j + 2, buf, sem)
                    cstart(j + 2, cbuf, csem)
            pl.when(j % 2 == 0)(lambda: step(rows0, sem0, col0, csem0))
            pl.when(j % 2 == 1)(lambda: step(rows1, sem1, col1, csem1))
            return carry

        lax.fori_loop(0, NB, body, 0)
        plsc.subcore_barrier()
        pltpu.sync_copy(accs.at[nsl_a], out_hbm.at[nsl_a])
        pl.when(s < NS - 1)(
            lambda: pltpu.sync_copy(accs.at[nsl_b], out_hbm.at[nsl_b]))

    pl.when(c == 0)(lambda: run(y0_hbm, out0_hbm))
    pl.when(c == 1)(lambda: run(y1_hbm, out1_hbm))


@functools.partial(
    pl.kernel,
    out_type=jax.ShapeDtypeStruct((NC * NS * N,), jnp.float32),
    mesh=_mesh,
    compiler_params=pltpu.CompilerParams(needs_layout_passes=False),
    scratch_types=[
        pltpu.VMEM((N,), jnp.float32),
        pltpu.VMEM((N,), jnp.float32),
        pltpu.VMEM((EDGES_PER_TILE_ALL + 16,), jnp.int32),
        pltpu.VMEM((EDGES_PER_TILE_ALL + 16,), jnp.int32),
    ],
)
def _prop1_kernel(y3_hbm, row_hbm, col_hbm, out_hbm, acc, y3v, rowbuf, colbuf):
    c = lax.axis_index("c")
    s = lax.axis_index("s")
    wid = s * NC + c
    pltpu.sync_copy(y3_hbm, y3v)
    zeros16 = jnp.zeros((16,), jnp.float32)

    def zbody(j, carry):
        acc[pl.ds(j * 16, 16)] = zeros16
        return carry

    lax.fori_loop(0, N // 16, zbody, 0)
    esl = pl.ds(wid * EDGES_PER_TILE_ALL, EDGES_PER_TILE_ALL)
    bsl = pl.ds(0, EDGES_PER_TILE_ALL)
    pltpu.sync_copy(row_hbm.at[esl], rowbuf.at[bsl])
    pltpu.sync_copy(col_hbm.at[esl], colbuf.at[bsl])
    lanes = lax.iota(jnp.int32, 16)
    nit = (EDGES_PER_TILE_ALL + 15) // 16

    def body(j, carry):
        ridx = jnp.minimum(jnp.maximum(rowbuf[pl.ds(j * 16, 16)], 0), N - 1)
        cidx = jnp.minimum(jnp.maximum(colbuf[pl.ds(j * 16, 16)], 0), N - 1)
        valid = lanes < (EDGES_PER_TILE_ALL - j * 16)
        vals = jnp.where(valid, plsc.load_gather(y3v, [ridx]), 0.0)
        plsc.addupdate_scatter(acc, [cidx], vals)
        return carry

    lax.fori_loop(0, nit, body, 0)
    pltpu.sync_copy(acc, out_hbm.at[pl.ds(wid * N, N)])


def _deg_reduce_body(p_ref, deg_ref):
    p = p_ref[...]
    ones = jnp.ones((p.shape[0], 1), jnp.float32)
    deg_ref[...] = lax.dot_general(
        p, ones, (((0,), (0,)), ((), ()))) + 1.0


_deg_reduce = pl.pallas_call(
    _deg_reduce_body,
    out_shape=jax.ShapeDtypeStruct((N, 1), jnp.float32),
)


def _mm1_body(x_ref, w_ref, deg_ref, y0_ref, y1_ref):
    dis = lax.rsqrt(deg_ref[...])
    y = jnp.dot(x_ref[...], w_ref[...],
                preferred_element_type=jnp.float32) * dis
    y0_ref[...] = y[:, :HALF]
    y1_ref[...] = y[:, HALF:]


_mm1 = pl.pallas_call(
    _mm1_body,
    grid=(GRID_M,),
    in_specs=[
        pl.BlockSpec((MBLK, D), lambda i: (i, 0)),
        pl.BlockSpec((D, D), lambda i: (0, 0)),
        pl.BlockSpec((MBLK, 1), lambda i: (i, 0)),
    ],
    out_specs=(
        pl.BlockSpec((MBLK, HALF), lambda i: (i, 0)),
        pl.BlockSpec((MBLK, HALF), lambda i: (i, 0)),
    ),
    out_shape=(
        jax.ShapeDtypeStruct((N, HALF), jnp.float32),
        jax.ShapeDtypeStruct((N, HALF), jnp.float32),
    ),
)


def _mm2_body(a0_ref, a1_ref, deg_ref, w_ref, b_ref, y0_ref, y1_ref):
    dis = lax.rsqrt(deg_ref[...])
    b = b_ref[...]
    h0 = jax.nn.sigmoid(a0_ref[...] * dis + b[:, :HALF])
    h1 = jax.nn.sigmoid(a1_ref[...] * dis + b[:, HALF:])
    h = jnp.concatenate([h0, h1], axis=1)
    y = jnp.dot(h, w_ref[...], preferred_element_type=jnp.float32) * dis
    y0_ref[...] = y[:, :HALF]
    y1_ref[...] = y[:, HALF:]


_mm2 = pl.pallas_call(
    _mm2_body,
    grid=(GRID_M,),
    in_specs=[
        pl.BlockSpec((MBLK, HALF), lambda i: (i, 0)),
        pl.BlockSpec((MBLK, HALF), lambda i: (i, 0)),
        pl.BlockSpec((MBLK, 1), lambda i: (i, 0)),
        pl.BlockSpec((D, D), lambda i: (0, 0)),
        pl.BlockSpec((1, D), lambda i: (0, 0)),
    ],
    out_specs=(
        pl.BlockSpec((MBLK, HALF), lambda i: (i, 0)),
        pl.BlockSpec((MBLK, HALF), lambda i: (i, 0)),
    ),
    out_shape=(
        jax.ShapeDtypeStruct((N, HALF), jnp.float32),
        jax.ShapeDtypeStruct((N, HALF), jnp.float32),
    ),
)


def _mm3_body(a0_ref, a1_ref, deg_ref, w_ref, b_ref, y3_ref):
    dis = lax.rsqrt(deg_ref[...])
    b = b_ref[...]
    h0 = jax.nn.sigmoid(a0_ref[...] * dis + b[:, :HALF])
    h1 = jax.nn.sigmoid(a1_ref[...] * dis + b[:, HALF:])
    h = jnp.concatenate([h0, h1], axis=1)
    y3_ref[...] = jnp.dot(h, w_ref[...],
                          preferred_element_type=jnp.float32) * dis


_mm3 = pl.pallas_call(
    _mm3_body,
    grid=(GRID_M,),
    in_specs=[
        pl.BlockSpec((MBLK, HALF), lambda i: (i, 0)),
        pl.BlockSpec((MBLK, HALF), lambda i: (i, 0)),
        pl.BlockSpec((MBLK, 1), lambda i: (i, 0)),
        pl.BlockSpec((D, 1), lambda i: (0, 0)),
        pl.BlockSpec((1, D), lambda i: (0, 0)),
    ],
    out_specs=pl.BlockSpec((MBLK, 1), lambda i: (i, 0)),
    out_shape=jax.ShapeDtypeStruct((N, 1), jnp.float32),
)


def _final_body(p_ref, y3_ref, deg_ref, b3_ref, w4_ref, b4_ref, w5_ref,
                b5_ref, out_ref):
    p = p_ref[...]
    ones = jnp.ones((p.shape[0], 1), jnp.float32)
    acc3 = lax.dot_general(p, ones, (((0,), (0,)), ((), ()))) + y3_ref[...]
    out3 = lax.rsqrt(deg_ref[...]) * acc3 + b3_ref[...]
    m = jnp.sum(out3) * (1.0 / N)
    r = (m * w4_ref[...] + b4_ref[...]) * w5_ref[...] + b5_ref[...]
    out_ref[...] = r


_final = pl.pallas_call(
    _final_body,
    out_shape=jax.ShapeDtypeStruct((1, 1), jnp.float32),
)


def kernel(node_features, edge_index, W1, b1, W2, b2, W3, b3, W4, b4, W5, b5):
    row = edge_index[0]
    col = edge_index[1]
    row3 = row.reshape(NS, NB, EB)
    partials = _deg_kernel(col).reshape(NC * NS, N)
    deg = _deg_reduce(partials)
    y10, y11 = _mm1(node_features, W1, deg)
    a10, a11 = _prop_kernel(y10, y11, row3, col)
    y20, y21 = _mm2(a10, a11, deg, W2, b1.reshape(1, D))
    a20, a21 = _prop_kernel(y20, y21, row3, col)
    y3 = _mm3(a20, a21, deg, W3, b2.reshape(1, D))
    p3 = _prop1_kernel(y3.reshape(N), row, col).reshape(NC * NS, N)
    out = _final(p3, y3, deg, b3.reshape(1, 1), W4, b4.reshape(1, 1),
                 W5, b5.reshape(1, 1))
    return out


# speedup vs baseline: 21.9657x; 1.0749x over previous
import functools

import jax
import jax.numpy as jnp
from jax import lax
from jax.experimental import pallas as pl
from jax.experimental.pallas import tpu as pltpu
from jax.experimental.pallas import tpu_sc as plsc

N = 10000
E = 160000
D = 256
HALF = 128
NC = 2
NS = 16
ROWS_PER_TILE = N // NS
RCHUNK = 632
RLAST = N - (NS - 1) * RCHUNK
EDGES_PER_TILE_SC = E // NS
EB = 125
EDGES_PER_TILE_ALL = E // (NC * NS)
MBLK = 1000
GRID_M = N // MBLK

_mesh = plsc.VectorSubcoreMesh(core_axis_name="c", subcore_axis_name="s")


@functools.partial(
    pl.kernel,
    out_type=jax.ShapeDtypeStruct((NC * NS * N,), jnp.float32),
    mesh=_mesh,
    compiler_params=pltpu.CompilerParams(needs_layout_passes=False),
    scratch_types=[
        pltpu.VMEM((N,), jnp.float32),
        pltpu.VMEM((EDGES_PER_TILE_ALL + 16,), jnp.int32),
    ],
)
def _deg_kernel(col_hbm, out_hbm, hist, colbuf):
    c = lax.axis_index("c")
    s = lax.axis_index("s")
    wid = s * NC + c
    zeros16 = jnp.zeros((16,), jnp.float32)

    def zbody(j, carry):
        hist[pl.ds(j * 16, 16)] = zeros16
        return carry

    lax.fori_loop(0, N // 16, zbody, 0)
    pltpu.sync_copy(
        col_hbm.at[pl.ds(wid * EDGES_PER_TILE_ALL, EDGES_PER_TILE_ALL)],
        colbuf.at[pl.ds(0, EDGES_PER_TILE_ALL)],
    )
    ones16 = jnp.ones((16,), jnp.float32)
    lanes = lax.iota(jnp.int32, 16)
    nit = (EDGES_PER_TILE_ALL + 15) // 16

    def body(j, carry):
        idx = colbuf[pl.ds(j * 16, 16)]
        idx = jnp.minimum(jnp.maximum(idx, 0), N - 1)
        valid = lanes < (EDGES_PER_TILE_ALL - j * 16)
        vals = jnp.where(valid, ones16, 0.0)
        plsc.addupdate_scatter(hist, [idx], vals)
        return carry

    lax.fori_loop(0, nit, body, 0)
    pltpu.sync_copy(hist, out_hbm.at[pl.ds(wid * N, N)])


NB = EDGES_PER_TILE_SC // EB


@functools.partial(
    pl.kernel,
    out_type=(
        jax.ShapeDtypeStruct((N, HALF), jnp.float32),
        jax.ShapeDtypeStruct((N, HALF), jnp.float32),
    ),
    mesh=_mesh,
    compiler_params=pltpu.CompilerParams(needs_layout_passes=False),
    scratch_types=[
        pltpu.VMEM_SHARED((N, HALF), jnp.float32),
        pltpu.VMEM((NB, EB), jnp.int32),
        pltpu.VMEM((1, EB), jnp.int32),
        pltpu.VMEM((1, EB), jnp.int32),
        pltpu.VMEM((EB, HALF), jnp.float32),
        pltpu.VMEM((EB, HALF), jnp.float32),
        pltpu.SemaphoreType.DMA,
        pltpu.SemaphoreType.DMA,
        pltpu.SemaphoreType.DMA,
        pltpu.SemaphoreType.DMA,
    ],
)
def _prop_kernel(y0_hbm, y1_hbm, row_hbm, col_hbm, out0_hbm, out1_hbm,
                 accs, rowbuf, col0, col1, rows0, rows1,
                 sem0, sem1, csem0, csem1):
    c = lax.axis_index("c")
    s = lax.axis_index("s")

    def run(y_hbm, out_hbm):
        base = s * RCHUNK
        nsl_a = pl.ds(base, RLAST)
        nsl_b = pl.ds(base + RLAST, RCHUNK - RLAST)
        pltpu.sync_copy(row_hbm.at[s], rowbuf)
        pltpu.sync_copy(y_hbm.at[nsl_a], accs.at[nsl_a])
        pl.when(s < NS - 1)(
            lambda: pltpu.sync_copy(y_hbm.at[nsl_b], accs.at[nsl_b]))
        plsc.subcore_barrier()

        def start(j, buf, sem):
            pltpu.async_copy(y_hbm.at[rowbuf.at[j]], buf, sem)

        def finish(buf, sem):
            pltpu.make_async_copy(y_hbm.at[rowbuf.at[0]], buf, sem).wait()

        def cstart(j, cbuf, csem):
            pltpu.async_copy(col_hbm.at[s, j], cbuf, csem)

        def cfinish(cbuf, csem):
            pltpu.make_async_copy(col_hbm.at[0, 0], cbuf, csem).wait()

        start(0, rows0, sem0)
        start(1, rows1, sem1)
        cstart(0, col0, csem0)
        cstart(1, col1, csem1)

        def body(j, carry):
            def step(buf, sem, cbuf, csem):
                cfinish(cbuf, csem)
                finish(buf, sem)
                pltpu.sync_copy(buf, accs.at[cbuf.at[0]], add=True)
                @pl.when(j < NB - 2)
                def _():
                    start(j + 2, buf, sem)
                    cstart(j + 2, cbuf, csem)
            pl.when(j % 2 == 0)(lambda: step(rows0, sem0, col0, csem0))
            pl.when(j % 2 == 1)(lambda: step(rows1, sem1, col1, csem1))
            return carry

        lax.fori_loop(0, NB, body, 0)
        plsc.subcore_barrier()
        pltpu.sync_copy(accs.at[nsl_a], out_hbm.at[nsl_a])
        pl.when(s < NS - 1)(
            lambda: pltpu.sync_copy(accs.at[nsl_b], out_hbm.at[nsl_b]))

    pl.when(c == 0)(lambda: run(y0_hbm, out0_hbm))
    pl.when(c == 1)(lambda: run(y1_hbm, out1_hbm))


@functools.partial(
    pl.kernel,
    out_type=jax.ShapeDtypeStruct((NC * NS * N,), jnp.float32),
    mesh=_mesh,
    compiler_params=pltpu.CompilerParams(needs_layout_passes=False),
    scratch_types=[
        pltpu.VMEM((N,), jnp.float32),
        pltpu.VMEM((N,), jnp.float32),
        pltpu.VMEM((EDGES_PER_TILE_ALL + 16,), jnp.int32),
        pltpu.VMEM((EDGES_PER_TILE_ALL + 16,), jnp.int32),
    ],
)
def _prop1_kernel(y3_hbm, row_hbm, col_hbm, out_hbm, acc, y3v, rowbuf, colbuf):
    c = lax.axis_index("c")
    s = lax.axis_index("s")
    wid = s * NC + c
    pltpu.sync_copy(y3_hbm, y3v)
    zeros16 = jnp.zeros((16,), jnp.float32)

    def zbody(j, carry):
        acc[pl.ds(j * 16, 16)] = zeros16
        return carry

    lax.fori_loop(0, N // 16, zbody, 0)
    esl = pl.ds(wid * EDGES_PER_TILE_ALL, EDGES_PER_TILE_ALL)
    bsl = pl.ds(0, EDGES_PER_TILE_ALL)
    pltpu.sync_copy(row_hbm.at[esl], rowbuf.at[bsl])
    pltpu.sync_copy(col_hbm.at[esl], colbuf.at[bsl])
    lanes = lax.iota(jnp.int32, 16)
    nit = (EDGES_PER_TILE_ALL + 15) // 16

    def body(j, carry):
        ridx = jnp.minimum(jnp.maximum(rowbuf[pl.ds(j * 16, 16)], 0), N - 1)
        cidx = jnp.minimum(jnp.maximum(colbuf[pl.ds(j * 16, 16)], 0), N - 1)
        valid = lanes < (EDGES_PER_TILE_ALL - j * 16)
        vals = jnp.where(valid, plsc.load_gather(y3v, [ridx]), 0.0)
        plsc.addupdate_scatter(acc, [cidx], vals)
        return carry

    lax.fori_loop(0, nit, body, 0)
    pltpu.sync_copy(acc, out_hbm.at[pl.ds(wid * N, N)])


def _deg_reduce_body(p_ref, deg_ref):
    p = p_ref[...]
    ones = jnp.ones((p.shape[0], 1), jnp.float32)
    deg_ref[...] = lax.dot_general(
        p, ones, (((0,), (0,)), ((), ()))) + 1.0


_deg_reduce = pl.pallas_call(
    _deg_reduce_body,
    out_shape=jax.ShapeDtypeStruct((N, 1), jnp.float32),
)


def _mm1_body(x_ref, w_ref, deg_ref, y0_ref, y1_ref):
    dis = lax.rsqrt(deg_ref[...])
    y = jnp.dot(x_ref[...], w_ref[...],
                preferred_element_type=jnp.float32) * dis
    y0_ref[...] = y[:, :HALF]
    y1_ref[...] = y[:, HALF:]


_mm1 = pl.pallas_call(
    _mm1_body,
    grid=(GRID_M,),
    in_specs=[
        pl.BlockSpec((MBLK, D), lambda i: (i, 0)),
        pl.BlockSpec((D, D), lambda i: (0, 0)),
        pl.BlockSpec((MBLK, 1), lambda i: (i, 0)),
    ],
    out_specs=(
        pl.BlockSpec((MBLK, HALF), lambda i: (i, 0)),
        pl.BlockSpec((MBLK, HALF), lambda i: (i, 0)),
    ),
    out_shape=(
        jax.ShapeDtypeStruct((N, HALF), jnp.float32),
        jax.ShapeDtypeStruct((N, HALF), jnp.float32),
    ),
)


def _mm2_body(a0_ref, a1_ref, deg_ref, w_ref, b_ref, y0_ref, y1_ref):
    dis = lax.rsqrt(deg_ref[...])
    b = b_ref[...]
    h0 = jax.nn.sigmoid(a0_ref[...] * dis + b[:, :HALF])
    h1 = jax.nn.sigmoid(a1_ref[...] * dis + b[:, HALF:])
    h = jnp.concatenate([h0, h1], axis=1)
    y = jnp.dot(h, w_ref[...], preferred_element_type=jnp.float32) * dis
    y0_ref[...] = y[:, :HALF]
    y1_ref[...] = y[:, HALF:]


_mm2 = pl.pallas_call(
    _mm2_body,
    grid=(GRID_M,),
    in_specs=[
        pl.BlockSpec((MBLK, HALF), lambda i: (i, 0)),
        pl.BlockSpec((MBLK, HALF), lambda i: (i, 0)),
        pl.BlockSpec((MBLK, 1), lambda i: (i, 0)),
        pl.BlockSpec((D, D), lambda i: (0, 0)),
        pl.BlockSpec((1, D), lambda i: (0, 0)),
    ],
    out_specs=(
        pl.BlockSpec((MBLK, HALF), lambda i: (i, 0)),
        pl.BlockSpec((MBLK, HALF), lambda i: (i, 0)),
    ),
    out_shape=(
        jax.ShapeDtypeStruct((N, HALF), jnp.float32),
        jax.ShapeDtypeStruct((N, HALF), jnp.float32),
    ),
)


def _mm3_body(a0_ref, a1_ref, deg_ref, w_ref, b_ref, y3_ref):
    dis = lax.rsqrt(deg_ref[...])
    b = b_ref[...]
    h0 = jax.nn.sigmoid(a0_ref[...] * dis + b[:, :HALF])
    h1 = jax.nn.sigmoid(a1_ref[...] * dis + b[:, HALF:])
    h = jnp.concatenate([h0, h1], axis=1)
    y3_ref[...] = jnp.dot(h, w_ref[...],
                          preferred_element_type=jnp.float32) * dis


_mm3 = pl.pallas_call(
    _mm3_body,
    grid=(GRID_M,),
    in_specs=[
        pl.BlockSpec((MBLK, HALF), lambda i: (i, 0)),
        pl.BlockSpec((MBLK, HALF), lambda i: (i, 0)),
        pl.BlockSpec((MBLK, 1), lambda i: (i, 0)),
        pl.BlockSpec((D, 1), lambda i: (0, 0)),
        pl.BlockSpec((1, D), lambda i: (0, 0)),
    ],
    out_specs=pl.BlockSpec((MBLK, 1), lambda i: (i, 0)),
    out_shape=jax.ShapeDtypeStruct((N, 1), jnp.float32),
)


def _final_body(p_ref, y3_ref, deg_ref, b3_ref, w4_ref, b4_ref, w5_ref,
                b5_ref, out_ref):
    p = p_ref[...]
    ones = jnp.ones((p.shape[0], 1), jnp.float32)
    acc3 = lax.dot_general(p, ones, (((0,), (0,)), ((), ()))) + y3_ref[...]
    out3 = lax.rsqrt(deg_ref[...]) * acc3 + b3_ref[...]
    m = jnp.sum(out3) * (1.0 / N)
    r = (m * w4_ref[...] + b4_ref[...]) * w5_ref[...] + b5_ref[...]
    out_ref[...] = r


_final = pl.pallas_call(
    _final_body,
    out_shape=jax.ShapeDtypeStruct((1, 1), jnp.float32),
)


def kernel(node_features, edge_index, W1, b1, W2, b2, W3, b3, W4, b4, W5, b5):
    row = edge_index[0]
    col = edge_index[1]
    row3 = row.reshape(NS, NB, EB)
    col4 = col.reshape(NS, NB, 1, EB)
    partials = _deg_kernel(col).reshape(NC * NS, N)
    deg = _deg_reduce(partials)
    y10, y11 = _mm1(node_features, W1, deg)
    a10, a11 = _prop_kernel(y10, y11, row3, col4)
    y20, y21 = _mm2(a10, a11, deg, W2, b1.reshape(1, D))
    a20, a21 = _prop_kernel(y20, y21, row3, col4)
    y3 = _mm3(a20, a21, deg, W3, b2.reshape(1, D))
    p3 = _prop1_kernel(y3.reshape(N), row, col).reshape(NC * NS, N)
    out = _final(p3, y3, deg, b3.reshape(1, 1), W4, b4.reshape(1, 1),
                 W5, b5.reshape(1, 1))
    return out


# speedup vs baseline: 22.9812x; 1.0462x over previous
import functools

import jax
import jax.numpy as jnp
from jax import lax
from jax.experimental import pallas as pl
from jax.experimental.pallas import tpu as pltpu
from jax.experimental.pallas import tpu_sc as plsc

N = 10000
E = 160000
D = 256
HALF = 128
NC = 2
NS = 16
ROWS_PER_TILE = N // NS
RCHUNK = 632
RLAST = N - (NS - 1) * RCHUNK
EDGES_PER_TILE_SC = E // NS
EB = 125
EDGES_PER_TILE_ALL = E // (NC * NS)
MBLK = 1000
GRID_M = N // MBLK

_mesh = plsc.VectorSubcoreMesh(core_axis_name="c", subcore_axis_name="s")


@functools.partial(
    pl.kernel,
    out_type=jax.ShapeDtypeStruct((NC * NS * N,), jnp.float32),
    mesh=_mesh,
    compiler_params=pltpu.CompilerParams(needs_layout_passes=False),
    scratch_types=[
        pltpu.VMEM((N,), jnp.float32),
        pltpu.VMEM((EDGES_PER_TILE_ALL + 16,), jnp.int32),
    ],
)
def _deg_kernel(col_hbm, out_hbm, hist, colbuf):
    c = lax.axis_index("c")
    s = lax.axis_index("s")
    wid = s * NC + c
    zeros16 = jnp.zeros((16,), jnp.float32)

    def zbody(j, carry):
        hist[pl.ds(j * 16, 16)] = zeros16
        return carry

    lax.fori_loop(0, N // 16, zbody, 0)
    pltpu.sync_copy(
        col_hbm.at[pl.ds(wid * EDGES_PER_TILE_ALL, EDGES_PER_TILE_ALL)],
        colbuf.at[pl.ds(0, EDGES_PER_TILE_ALL)],
    )
    ones16 = jnp.ones((16,), jnp.float32)
    lanes = lax.iota(jnp.int32, 16)
    nit = (EDGES_PER_TILE_ALL + 15) // 16

    def body(j, carry):
        idx = colbuf[pl.ds(j * 16, 16)]
        idx = jnp.minimum(jnp.maximum(idx, 0), N - 1)
        valid = lanes < (EDGES_PER_TILE_ALL - j * 16)
        vals = jnp.where(valid, ones16, 0.0)
        plsc.addupdate_scatter(hist, [idx], vals)
        return carry

    lax.fori_loop(0, nit, body, 0)
    pltpu.sync_copy(hist, out_hbm.at[pl.ds(wid * N, N)])


NB = EDGES_PER_TILE_SC // EB


@functools.partial(
    pl.kernel,
    out_type=(
        jax.ShapeDtypeStruct((N, HALF), jnp.float32),
        jax.ShapeDtypeStruct((N, HALF), jnp.float32),
    ),
    mesh=_mesh,
    compiler_params=pltpu.CompilerParams(needs_layout_passes=False),
    scratch_types=[
        pltpu.VMEM_SHARED((N, HALF), jnp.float32),
        pltpu.VMEM((NB, EB), jnp.int32),
        pltpu.VMEM((1, EB), jnp.int32),
        pltpu.VMEM((1, EB), jnp.int32),
        pltpu.VMEM((EB, HALF), jnp.float32),
        pltpu.VMEM((EB, HALF), jnp.float32),
        pltpu.SemaphoreType.DMA,
        pltpu.SemaphoreType.DMA,
        pltpu.SemaphoreType.DMA,
        pltpu.SemaphoreType.DMA,
    ],
)
def _prop_kernel(y0_hbm, y1_hbm, row_hbm, col_hbm, out0_hbm, out1_hbm,
                 accs, rowbuf, col0, col1, rows0, rows1,
                 sem0, sem1, csem0, csem1):
    c = lax.axis_index("c")
    s = lax.axis_index("s")

    def run(y_hbm, out_hbm):
        base = s * RCHUNK
        nsl_a = pl.ds(base, RLAST)
        nsl_b = pl.ds(base + RLAST, RCHUNK - RLAST)
        pltpu.sync_copy(row_hbm.at[s], rowbuf)
        pltpu.sync_copy(y_hbm.at[nsl_a], accs.at[nsl_a])
        pl.when(s < NS - 1)(
            lambda: pltpu.sync_copy(y_hbm.at[nsl_b], accs.at[nsl_b]))
        plsc.subcore_barrier()

        def start(j, buf, sem):
            pltpu.async_copy(y_hbm.at[rowbuf.at[j]], buf, sem)

        def finish(buf, sem):
            pltpu.make_async_copy(y_hbm.at[rowbuf.at[0]], buf, sem).wait()

        def cstart(j, cbuf, csem):
            pltpu.async_copy(col_hbm.at[s, j], cbuf, csem)

        def cfinish(cbuf, csem):
            pltpu.make_async_copy(col_hbm.at[0, 0], cbuf, csem).wait()

        start(0, rows0, sem0)
        start(1, rows1, sem1)
        cstart(0, col0, csem0)
        cstart(1, col1, csem1)

        def body(j, carry):
            def step(buf, sem, cbuf, csem):
                cfinish(cbuf, csem)
                finish(buf, sem)
                pltpu.sync_copy(buf, accs.at[cbuf.at[0]], add=True)
                @pl.when(j < NB - 2)
                def _():
                    start(j + 2, buf, sem)
                    cstart(j + 2, cbuf, csem)
            pl.when(j % 2 == 0)(lambda: step(rows0, sem0, col0, csem0))
            pl.when(j % 2 == 1)(lambda: step(rows1, sem1, col1, csem1))
            return carry

        lax.fori_loop(0, NB, body, 0)
        plsc.subcore_barrier()
        pltpu.sync_copy(accs.at[nsl_a], out_hbm.at[nsl_a])
        pl.when(s < NS - 1)(
            lambda: pltpu.sync_copy(accs.at[nsl_b], out_hbm.at[nsl_b]))

    pl.when(c == 0)(lambda: run(y0_hbm, out0_hbm))
    pl.when(c == 1)(lambda: run(y1_hbm, out1_hbm))


@functools.partial(
    pl.kernel,
    out_type=jax.ShapeDtypeStruct((NC * NS * N,), jnp.float32),
    mesh=_mesh,
    compiler_params=pltpu.CompilerParams(needs_layout_passes=False),
    scratch_types=[
        pltpu.VMEM((N,), jnp.float32),
        pltpu.VMEM((N,), jnp.float32),
        pltpu.VMEM((EDGES_PER_TILE_ALL + 16,), jnp.int32),
        pltpu.VMEM((EDGES_PER_TILE_ALL + 16,), jnp.int32),
    ],
)
def _q_kernel(dis_hbm, row_hbm, col_hbm, out_hbm, acc, disv, rowbuf, colbuf):
    c = lax.axis_index("c")
    s = lax.axis_index("s")
    wid = s * NC + c
    pltpu.sync_copy(dis_hbm, disv)
    zeros16 = jnp.zeros((16,), jnp.float32)

    def zbody(j, carry):
        acc[pl.ds(j * 16, 16)] = zeros16
        return carry

    lax.fori_loop(0, N // 16, zbody, 0)
    esl = pl.ds(wid * EDGES_PER_TILE_ALL, EDGES_PER_TILE_ALL)
    bsl = pl.ds(0, EDGES_PER_TILE_ALL)
    pltpu.sync_copy(row_hbm.at[esl], rowbuf.at[bsl])
    pltpu.sync_copy(col_hbm.at[esl], colbuf.at[bsl])
    lanes = lax.iota(jnp.int32, 16)
    nit = (EDGES_PER_TILE_ALL + 15) // 16

    def body(j, carry):
        ridx = jnp.minimum(jnp.maximum(rowbuf[pl.ds(j * 16, 16)], 0), N - 1)
        cidx = jnp.minimum(jnp.maximum(colbuf[pl.ds(j * 16, 16)], 0), N - 1)
        valid = lanes < (EDGES_PER_TILE_ALL - j * 16)
        vals = jnp.where(valid, plsc.load_gather(disv, [cidx]), 0.0)
        plsc.addupdate_scatter(acc, [ridx], vals)
        return carry

    lax.fori_loop(0, nit, body, 0)
    pltpu.sync_copy(acc, out_hbm.at[pl.ds(wid * N, N)])


def _deg_reduce_body(p_ref, dis_ref):
    p = p_ref[...]
    ones = jnp.ones((p.shape[0], 1), jnp.float32)
    deg = lax.dot_general(p, ones, (((0,), (0,)), ((), ()))) + 1.0
    dis_ref[...] = lax.rsqrt(deg)


_deg_reduce = pl.pallas_call(
    _deg_reduce_body,
    out_shape=jax.ShapeDtypeStruct((N, 1), jnp.float32),
)


def _mm1_body(x_ref, w_ref, dis_ref, y0_ref, y1_ref):
    dis = dis_ref[...]
    y = jnp.dot(x_ref[...], w_ref[...],
                preferred_element_type=jnp.float32) * dis
    y0_ref[...] = y[:, :HALF]
    y1_ref[...] = y[:, HALF:]


_mm1 = pl.pallas_call(
    _mm1_body,
    grid=(GRID_M,),
    in_specs=[
        pl.BlockSpec((MBLK, D), lambda i: (i, 0)),
        pl.BlockSpec((D, D), lambda i: (0, 0)),
        pl.BlockSpec((MBLK, 1), lambda i: (i, 0)),
    ],
    out_specs=(
        pl.BlockSpec((MBLK, HALF), lambda i: (i, 0)),
        pl.BlockSpec((MBLK, HALF), lambda i: (i, 0)),
    ),
    out_shape=(
        jax.ShapeDtypeStruct((N, HALF), jnp.float32),
        jax.ShapeDtypeStruct((N, HALF), jnp.float32),
    ),
)


def _mm2_body(a0_ref, a1_ref, dis_ref, w_ref, b_ref, y0_ref, y1_ref):
    dis = dis_ref[...]
    b = b_ref[...]
    h0 = jax.nn.sigmoid(a0_ref[...] * dis + b[:, :HALF])
    h1 = jax.nn.sigmoid(a1_ref[...] * dis + b[:, HALF:])
    h = jnp.concatenate([h0, h1], axis=1)
    y = jnp.dot(h, w_ref[...], preferred_element_type=jnp.float32) * dis
    y0_ref[...] = y[:, :HALF]
    y1_ref[...] = y[:, HALF:]


_mm2 = pl.pallas_call(
    _mm2_body,
    grid=(GRID_M,),
    in_specs=[
        pl.BlockSpec((MBLK, HALF), lambda i: (i, 0)),
        pl.BlockSpec((MBLK, HALF), lambda i: (i, 0)),
        pl.BlockSpec((MBLK, 1), lambda i: (i, 0)),
        pl.BlockSpec((D, D), lambda i: (0, 0)),
        pl.BlockSpec((1, D), lambda i: (0, 0)),
    ],
    out_specs=(
        pl.BlockSpec((MBLK, HALF), lambda i: (i, 0)),
        pl.BlockSpec((MBLK, HALF), lambda i: (i, 0)),
    ),
    out_shape=(
        jax.ShapeDtypeStruct((N, HALF), jnp.float32),
        jax.ShapeDtypeStruct((N, HALF), jnp.float32),
    ),
)


def _mm3_body(a0_ref, a1_ref, dis_ref, w_ref, b_ref, y3_ref):
    dis = dis_ref[...]
    b = b_ref[...]
    h0 = jax.nn.sigmoid(a0_ref[...] * dis + b[:, :HALF])
    h1 = jax.nn.sigmoid(a1_ref[...] * dis + b[:, HALF:])
    h = jnp.concatenate([h0, h1], axis=1)
    y3_ref[...] = jnp.dot(h, w_ref[...],
                          preferred_element_type=jnp.float32) * dis


_mm3 = pl.pallas_call(
    _mm3_body,
    grid=(GRID_M,),
    in_specs=[
        pl.BlockSpec((MBLK, HALF), lambda i: (i, 0)),
        pl.BlockSpec((MBLK, HALF), lambda i: (i, 0)),
        pl.BlockSpec((MBLK, 1), lambda i: (i, 0)),
        pl.BlockSpec((D, 1), lambda i: (0, 0)),
        pl.BlockSpec((1, D), lambda i: (0, 0)),
    ],
    out_specs=pl.BlockSpec((MBLK, 1), lambda i: (i, 0)),
    out_shape=jax.ShapeDtypeStruct((N, 1), jnp.float32),
)


def _final_body(p_ref, y3_ref, dis_ref, b3_ref, w4_ref, b4_ref, w5_ref,
                b5_ref, out_ref):
    p = p_ref[...]
    ones = jnp.ones((p.shape[0], 1), jnp.float32)
    q = lax.dot_general(p, ones, (((0,), (0,)), ((), ())))
    m = jnp.sum(y3_ref[...] * (dis_ref[...] + q)) * (1.0 / N) + b3_ref[...]
    r = (m * w4_ref[...] + b4_ref[...]) * w5_ref[...] + b5_ref[...]
    out_ref[...] = r


_final = pl.pallas_call(
    _final_body,
    out_shape=jax.ShapeDtypeStruct((1, 1), jnp.float32),
)


def kernel(node_features, edge_index, W1, b1, W2, b2, W3, b3, W4, b4, W5, b5):
    row = edge_index[0]
    col = edge_index[1]
    row3 = row.reshape(NS, NB, EB)
    col4 = col.reshape(NS, NB, 1, EB)
    partials = _deg_kernel(col).reshape(NC * NS, N)
    dis = _deg_reduce(partials)
    qp = _q_kernel(dis.reshape(N), row, col).reshape(NC * NS, N)
    y10, y11 = _mm1(node_features, W1, dis)
    a10, a11 = _prop_kernel(y10, y11, row3, col4)
    y20, y21 = _mm2(a10, a11, dis, W2, b1.reshape(1, D))
    a20, a21 = _prop_kernel(y20, y21, row3, col4)
    y3 = _mm3(a20, a21, dis, W3, b2.reshape(1, D))
    out = _final(qp, y3, dis, b3.reshape(1, 1), W4, b4.reshape(1, 1),
                 W5, b5.reshape(1, 1))
    return out
